# baseline (speedup 1.0000x reference)
import functools

import jax
import jax.numpy as jnp
from jax import lax
from jax.experimental import pallas as pl
from jax.experimental.pallas import tpu as pltpu
from jax.experimental.pallas import tpu_sc as plsc

_B = 16384
_D = 64
_LAT = 128
_NC = 2
_NS = 16
_NW = _NC * _NS
_BPW = _B // _NW
_CHUNK = 128
_NCH = _BPW // _CHUNK
_NGB = 8
_NV2 = 500000


def _ncf_body(uidx_hbm, iidx_hbm, uT_hbm, iT_hbm, uA_hbm, iA_hbm, par_hbm,
              out_hbm,
              uidx_v, iidx_v, gidx_v, rows_v, uA_v, iA_v, par_v,
              wu_v, wi_v, out_v, sem):
    wid = lax.axis_index("s") * _NC + lax.axis_index("c")
    base = wid * _BPW

    for j in range(_NCH):
        pltpu.sync_copy(uidx_hbm.at[pl.ds(base + j * _CHUNK, _CHUNK)],
                        uidx_v.at[j])
        pltpu.sync_copy(iidx_hbm.at[pl.ds(base + j * _CHUNK, _CHUNK)],
                        iidx_v.at[j])

    for j in range(_NCH):
        for k in range(_CHUNK // 16):
            s = pl.ds(k * 16, 16)
            gidx_v[j, s] = lax.shift_right_logical(uidx_v[j, s], 1)

    copies = [pltpu.async_copy(
        uT_hbm.at[gidx_v.at[j]], rows_v.at[pl.ds(j * _CHUNK, _CHUNK)], sem)
        for j in range(_NCH)]

    pltpu.sync_copy(uA_hbm, uA_v)
    pltpu.sync_copy(iA_hbm, iA_v)
    pltpu.sync_copy(par_hbm, par_v)

    kidx = [lax.iota(jnp.int32, 16) + kc * 16 for kc in range(_D // 16)]

    def fold_body(l, carry):
        accs = list(carry)
        col = jnp.full((16,), l, dtype=jnp.int32)
        wlu = plsc.load_gather(par_v, [col])
        wli = plsc.load_gather(par_v, [col + _LAT])
        for kc in range(_D // 16):
            accs[kc] = accs[kc] + plsc.load_gather(uA_v, [kidx[kc], col]) * wlu
            accs[4 + kc] = (accs[4 + kc]
                            + plsc.load_gather(iA_v, [kidx[kc], col]) * wli)
        return tuple(accs)

    zero = jnp.zeros((16,), jnp.float32)
    accs = lax.fori_loop(0, _LAT, fold_body, (zero,) * 8)
    for kc in range(_D // 16):
        wu_v[pl.ds(kc * 16, 16)] = accs[kc]
        wi_v[pl.ds(kc * 16, 16)] = accs[4 + kc]

    bias = plsc.load_gather(par_v, [jnp.full((16,), 2 * _LAT, jnp.int32)])
    iota16 = lax.iota(jnp.int32, 16)

    def make_dot(idx_ref, w_ref, first):
        def dot_body(gb, carry):
            cbase = []
            for g in range(_NGB):
                e = gb * _NGB + g
                vi = idx_ref[e * 16 // _CHUNK, pl.ds((e * 16) % _CHUNK, 16)]
                cbase.append((vi & 1) * 64)
            accs = [zero] * _NGB
            bidxs = [iota16 + (gb * _NGB + g) * 16 for g in range(_NGB)]
            for d in range(_D):
                col = jnp.full((16,), d, dtype=jnp.int32)
                wl = plsc.load_gather(w_ref, [col])
                for g in range(_NGB):
                    accs[g] = accs[g] + plsc.load_gather(
                        rows_v, [bidxs[g], cbase[g] + col]) * wl
            for g in range(_NGB):
                s = pl.ds((gb * _NGB + g) * 16, 16)
                if first:
                    out_v[s] = accs[g] + bias
                else:
                    out_v[s] = out_v[s] + accs[g]
            return carry
        return dot_body

    for c in copies:
        c.wait()
    lax.fori_loop(0, _BPW // 16 // _NGB, make_dot(uidx_v, wu_v, True), 0)

    for j in range(_NCH):
        for k in range(_CHUNK // 16):
            s = pl.ds(k * 16, 16)
            gidx_v[j, s] = lax.shift_right_logical(iidx_v[j, s], 1)
    copies = [pltpu.async_copy(
        iT_hbm.at[gidx_v.at[j]], rows_v.at[pl.ds(j * _CHUNK, _CHUNK)], sem)
        for j in range(_NCH)]
    for c in copies:
        c.wait()
    lax.fori_loop(0, _BPW // 16 // _NGB, make_dot(iidx_v, wi_v, False), 0)

    pltpu.sync_copy(out_v, out_hbm.at[pl.ds(base, _BPW)])


@jax.jit
def _ncf(user_indices, item_indices, user_T2, item_T2, user_A, item_A,
         params):
    run = pl.kernel(
        _ncf_body,
        out_type=jax.ShapeDtypeStruct((_B,), jnp.float32),
        mesh=plsc.VectorSubcoreMesh(core_axis_name="c", subcore_axis_name="s"),
        compiler_params=pltpu.CompilerParams(needs_layout_passes=False,
                                             use_tc_tiling_on_sc=True),
        scratch_types=[
            pltpu.VMEM((_NCH, _CHUNK), jnp.int32),
            pltpu.VMEM((_NCH, _CHUNK), jnp.int32),
            pltpu.VMEM((_NCH, _CHUNK), jnp.int32),
            pltpu.VMEM((_BPW, _LAT), jnp.float32),
            pltpu.VMEM((_D, _LAT), jnp.float32),
            pltpu.VMEM((_D, _LAT), jnp.float32),
            pltpu.VMEM((264,), jnp.float32),
            pltpu.VMEM((_D,), jnp.float32),
            pltpu.VMEM((_D,), jnp.float32),
            pltpu.VMEM((_BPW,), jnp.float32),
            pltpu.SemaphoreType.DMA,
        ],
    )
    return run(user_indices, item_indices, user_T2, item_T2, user_A, item_A,
               params)


def kernel(user_indices, item_indices, user_T, item_T, user_A, item_A,
           W_aff, b_aff):
    params = jnp.concatenate([W_aff.reshape(-1), b_aff.reshape(-1),
                              jnp.zeros((7,), jnp.float32)])
    out = _ncf(user_indices.astype(jnp.int32), item_indices.astype(jnp.int32),
               user_T.reshape(_NV2, _LAT), item_T.reshape(_NV2, _LAT),
               user_A, item_A, params)
    return out.reshape(_B, 1)


# speedup vs baseline: 2.9883x; 2.9883x over previous
import functools

import jax
import jax.numpy as jnp
from jax import lax
from jax.experimental import pallas as pl
from jax.experimental.pallas import tpu as pltpu
from jax.experimental.pallas import tpu_sc as plsc

_B = 16384
_N = 1000000
_D = 64
_LAT = 128
_NC = 2
_NS = 16
_NW = _NC * _NS
_BPW = _B // _NW
_CHUNK = 128
_NCH = _BPW // _CHUNK
_W = 4096
_VROWS = 8192

_HI = jax.lax.Precision.HIGHEST


def _mv_body(uT_ref, iT_ref, uA_ref, iA_ref, W_ref, vu_ref, vi_ref):
    w = W_ref[...]
    wu = jax.lax.dot_general(uA_ref[...], w[0:_LAT, :],
                             (((1,), (0,)), ((), ())), precision=_HI)
    wi = jax.lax.dot_general(iA_ref[...], w[_LAT:2 * _LAT, :],
                             (((1,), (0,)), ((), ())), precision=_HI)
    pu = jax.lax.dot_general(wu.reshape(1, _D), uT_ref[...],
                             (((1,), (0,)), ((), ())), precision=_HI)
    pi = jax.lax.dot_general(wi.reshape(1, _D), iT_ref[...],
                             (((1,), (0,)), ((), ())), precision=_HI)
    vu_ref[...] = pu.reshape(_W // 128, 128)
    vi_ref[...] = pi.reshape(_W // 128, 128)


def _gather_body(uidx_hbm, iidx_hbm, vu_hbm, vi_hbm, par_hbm, out_hbm,
                 uidx_v, iidx_v, gidx_v, rows_v, par_v, out_v, sem):
    wid = lax.axis_index("s") * _NC + lax.axis_index("c")
    base = wid * _BPW

    for j in range(_NCH):
        pltpu.sync_copy(uidx_hbm.at[pl.ds(base + j * _CHUNK, _CHUNK)],
                        uidx_v.at[j])
        pltpu.sync_copy(iidx_hbm.at[pl.ds(base + j * _CHUNK, _CHUNK)],
                        iidx_v.at[j])
    pltpu.sync_copy(par_hbm, par_v)

    bias = plsc.load_gather(par_v, [jnp.full((16,), 2 * _LAT, jnp.int32)])
    iota16 = lax.iota(jnp.int32, 16)

    def one_table(v_hbm, idx_ref, first):
        for j in range(_NCH):
            for k in range(_CHUNK // 16):
                s = pl.ds(k * 16, 16)
                gidx_v[j, s] = lax.shift_right_logical(idx_ref[j, s], 7)
        copies = [pltpu.async_copy(
            v_hbm.at[gidx_v.at[j]], rows_v.at[pl.ds(j * _CHUNK, _CHUNK)], sem)
            for j in range(_NCH)]
        for c in copies:
            c.wait()

        def group(g, carry):
            vi = idx_ref[g >> 3, pl.ds((g & 7) * 16, 16)]
            lane = vi & 127
            val = plsc.load_gather(rows_v, [iota16 + g * 16, lane])
            s = pl.ds(g * 16, 16)
            if first:
                out_v[s] = val + bias
            else:
                out_v[s] = out_v[s] + val
            return carry

        lax.fori_loop(0, _BPW // 16, group, 0)

    one_table(vu_hbm, uidx_v, True)
    one_table(vi_hbm, iidx_v, False)

    pltpu.sync_copy(out_v, out_hbm.at[pl.ds(base, _BPW)])


@jax.jit
def _ncf(user_indices, item_indices, user_Tt, item_Tt, user_A, item_A,
         W_aff, params):
    grid = pl.cdiv(_N, _W)
    vu, vi = pl.pallas_call(
        _mv_body,
        grid=(grid,),
        in_specs=[
            pl.BlockSpec((_D, _W), lambda g: (0, g)),
            pl.BlockSpec((_D, _W), lambda g: (0, g)),
            pl.BlockSpec((_D, _LAT), lambda g: (0, 0)),
            pl.BlockSpec((_D, _LAT), lambda g: (0, 0)),
            pl.BlockSpec((2 * _LAT, 1), lambda g: (0, 0)),
        ],
        out_specs=[
            pl.BlockSpec((_W // 128, 128), lambda g: (g, 0)),
            pl.BlockSpec((_W // 128, 128), lambda g: (g, 0)),
        ],
        out_shape=[
            jax.ShapeDtypeStruct((_VROWS, 128), jnp.float32),
            jax.ShapeDtypeStruct((_VROWS, 128), jnp.float32),
        ],
    )(user_Tt, item_Tt, user_A, item_A, W_aff)

    run = pl.kernel(
        _gather_body,
        out_type=jax.ShapeDtypeStruct((_B,), jnp.float32),
        mesh=plsc.VectorSubcoreMesh(core_axis_name="c", subcore_axis_name="s"),
        compiler_params=pltpu.CompilerParams(needs_layout_passes=False,
                                             use_tc_tiling_on_sc=True),
        scratch_types=[
            pltpu.VMEM((_NCH, _CHUNK), jnp.int32),
            pltpu.VMEM((_NCH, _CHUNK), jnp.int32),
            pltpu.VMEM((_NCH, _CHUNK), jnp.int32),
            pltpu.VMEM((_BPW, 128), jnp.float32),
            pltpu.VMEM((264,), jnp.float32),
            pltpu.VMEM((_BPW,), jnp.float32),
            pltpu.SemaphoreType.DMA,
        ],
    )
    return run(user_indices, item_indices, vu, vi, params)


def kernel(user_indices, item_indices, user_T, item_T, user_A, item_A,
           W_aff, b_aff):
    params = jnp.concatenate([W_aff.reshape(-1), b_aff.reshape(-1),
                              jnp.zeros((7,), jnp.float32)])
    out = _ncf(user_indices.astype(jnp.int32), item_indices.astype(jnp.int32),
               user_T.T, item_T.T, user_A, item_A, W_aff, params)
    return out.reshape(_B, 1)


# speedup vs baseline: 5.5763x; 1.8660x over previous
import functools

import jax
import jax.numpy as jnp
from jax import lax
from jax.experimental import pallas as pl
from jax.experimental.pallas import tpu as pltpu
from jax.experimental.pallas import tpu_sc as plsc

_B = 16384
_N = 1000000
_D = 64
_LAT = 128
_NC = 2
_NS = 16
_NW = _NC * _NS
_BPW = _B // _NW
_CHUNK = 128
_NCH = _BPW // _CHUNK
_W = 8192
_VROWS = 8192

_HI = jax.lax.Precision.HIGHEST


def _mv_body(uT_ref, iT_ref, uA_ref, iA_ref, W_ref, vu_ref, vi_ref, w_s):
    @pl.when(pl.program_id(0) == 0)
    def _():
        w = W_ref[...]
        wu = jax.lax.dot_general(uA_ref[...], w[0:_LAT, :],
                                 (((1,), (0,)), ((), ())), precision=_HI)
        wi = jax.lax.dot_general(iA_ref[...], w[_LAT:2 * _LAT, :],
                                 (((1,), (0,)), ((), ())), precision=_HI)
        w_s[0:1, :] = wu.reshape(1, _D)
        w_s[1:2, :] = wi.reshape(1, _D)

    pu = jax.lax.dot_general(w_s[0:1, :], uT_ref[...],
                             (((1,), (0,)), ((), ())))
    pi = jax.lax.dot_general(w_s[1:2, :], iT_ref[...],
                             (((1,), (0,)), ((), ())))
    vu_ref[...] = pu.reshape(_W // 128, 128)
    vi_ref[...] = pi.reshape(_W // 128, 128)


def _gather_body(uidx_hbm, iidx_hbm, vu_hbm, vi_hbm, par_hbm, out_hbm,
                 uidx_v, iidx_v, gidx_v, rows_v, par_v, out_v, sem):
    wid = lax.axis_index("s") * _NC + lax.axis_index("c")
    base = wid * _BPW

    for j in range(_NCH):
        pltpu.sync_copy(uidx_hbm.at[pl.ds(base + j * _CHUNK, _CHUNK)],
                        uidx_v.at[j])
        pltpu.sync_copy(iidx_hbm.at[pl.ds(base + j * _CHUNK, _CHUNK)],
                        iidx_v.at[j])
    pltpu.sync_copy(par_hbm, par_v)

    bias = plsc.load_gather(par_v, [jnp.full((16,), 2 * _LAT, jnp.int32)])
    iota16 = lax.iota(jnp.int32, 16)

    def one_table(v_hbm, idx_ref, first):
        for j in range(_NCH):
            for k in range(_CHUNK // 16):
                s = pl.ds(k * 16, 16)
                gidx_v[j, s] = lax.shift_right_logical(idx_ref[j, s], 7)
        copies = [pltpu.async_copy(
            v_hbm.at[gidx_v.at[j]], rows_v.at[pl.ds(j * _CHUNK, _CHUNK)], sem)
            for j in range(_NCH)]
        for c in copies:
            c.wait()

        def group(g, carry):
            vi = idx_ref[g >> 3, pl.ds((g & 7) * 16, 16)]
            lane = vi & 127
            val = plsc.load_gather(rows_v, [iota16 + g * 16, lane])
            s = pl.ds(g * 16, 16)
            if first:
                out_v[s] = val + bias
            else:
                out_v[s] = out_v[s] + val
            return carry

        lax.fori_loop(0, _BPW // 16, group, 0)

    one_table(vu_hbm, uidx_v, True)
    one_table(vi_hbm, iidx_v, False)

    pltpu.sync_copy(out_v, out_hbm.at[pl.ds(base, _BPW)])


@jax.jit
def _ncf(user_indices, item_indices, user_Tt, item_Tt, user_A, item_A,
         W_aff, params):
    grid = pl.cdiv(_N, _W)
    vu, vi = pl.pallas_call(
        _mv_body,
        grid=(grid,),
        in_specs=[
            pl.BlockSpec((_D, _W), lambda g: (0, g)),
            pl.BlockSpec((_D, _W), lambda g: (0, g)),
            pl.BlockSpec((_D, _LAT), lambda g: (0, 0)),
            pl.BlockSpec((_D, _LAT), lambda g: (0, 0)),
            pl.BlockSpec((2 * _LAT, 1), lambda g: (0, 0)),
        ],
        out_specs=[
            pl.BlockSpec((_W // 128, 128), lambda g: (g, 0)),
            pl.BlockSpec((_W // 128, 128), lambda g: (g, 0)),
        ],
        out_shape=[
            jax.ShapeDtypeStruct((_VROWS, 128), jnp.float32),
            jax.ShapeDtypeStruct((_VROWS, 128), jnp.float32),
        ],
        scratch_shapes=[pltpu.VMEM((2, _D), jnp.float32)],
    )(user_Tt, item_Tt, user_A, item_A, W_aff)

    run = pl.kernel(
        _gather_body,
        out_type=jax.ShapeDtypeStruct((_B,), jnp.float32),
        mesh=plsc.VectorSubcoreMesh(core_axis_name="c", subcore_axis_name="s"),
        compiler_params=pltpu.CompilerParams(needs_layout_passes=False,
                                             use_tc_tiling_on_sc=True),
        scratch_types=[
            pltpu.VMEM((_NCH, _CHUNK), jnp.int32),
            pltpu.VMEM((_NCH, _CHUNK), jnp.int32),
            pltpu.VMEM((_NCH, _CHUNK), jnp.int32),
            pltpu.VMEM((_BPW, 128), jnp.float32),
            pltpu.VMEM((264,), jnp.float32),
            pltpu.VMEM((_BPW,), jnp.float32),
            pltpu.SemaphoreType.DMA,
        ],
    )
    return run(user_indices, item_indices, vu, vi, params)


def kernel(user_indices, item_indices, user_T, item_T, user_A, item_A,
           W_aff, b_aff):
    params = jnp.concatenate([W_aff.reshape(-1), b_aff.reshape(-1),
                              jnp.zeros((7,), jnp.float32)])
    out = _ncf(user_indices.astype(jnp.int32), item_indices.astype(jnp.int32),
               user_T.T, item_T.T, user_A, item_A, W_aff, params)
    return out.reshape(_B, 1)


# speedup vs baseline: 6.2720x; 1.1248x over previous
import functools

import jax
import jax.numpy as jnp
from jax import lax
from jax.experimental import pallas as pl
from jax.experimental.pallas import tpu as pltpu
from jax.experimental.pallas import tpu_sc as plsc

_B = 16384
_N = 1000000
_D = 64
_LAT = 128
_NC = 2
_NS = 16
_NW = _NC * _NS
_BPW = _B // _NW
_CHUNK = 128
_NCH = _BPW // _CHUNK
_W = 16384
_VROWS = 8192

_HI = jax.lax.Precision.HIGHEST


def _mv_body(uT_ref, iT_ref, uA_ref, iA_ref, W_ref, vu_ref, vi_ref, w_s):
    @pl.when(pl.program_id(0) == 0)
    def _():
        w = W_ref[...]
        wu = jax.lax.dot_general(uA_ref[...], w[0:_LAT, :],
                                 (((1,), (0,)), ((), ())), precision=_HI)
        wi = jax.lax.dot_general(iA_ref[...], w[_LAT:2 * _LAT, :],
                                 (((1,), (0,)), ((), ())), precision=_HI)
        w_s[0:1, :] = wu.reshape(1, _D)
        w_s[1:2, :] = wi.reshape(1, _D)

    pu = jax.lax.dot_general(w_s[0:1, :], uT_ref[...],
                             (((1,), (0,)), ((), ())))
    pi = jax.lax.dot_general(w_s[1:2, :], iT_ref[...],
                             (((1,), (0,)), ((), ())))
    vu_ref[...] = pu.reshape(_W // 128, 128)
    vi_ref[...] = pi.reshape(_W // 128, 128)


def _gather_body(uidx_hbm, iidx_hbm, vu_hbm, vi_hbm, par_hbm, out_hbm,
                 uidx_v, iidx_v, gidx_v, rows_v, par_v, out_v, sem):
    wid = lax.axis_index("s") * _NC + lax.axis_index("c")
    base = wid * _BPW

    for j in range(_NCH):
        pltpu.sync_copy(uidx_hbm.at[pl.ds(base + j * _CHUNK, _CHUNK)],
                        uidx_v.at[j])
        pltpu.sync_copy(iidx_hbm.at[pl.ds(base + j * _CHUNK, _CHUNK)],
                        iidx_v.at[j])
    pltpu.sync_copy(par_hbm, par_v)

    bias = plsc.load_gather(par_v, [jnp.full((16,), 2 * _LAT, jnp.int32)])
    iota16 = lax.iota(jnp.int32, 16)

    def one_table(v_hbm, idx_ref, first):
        for j in range(_NCH):
            for k in range(_CHUNK // 16):
                s = pl.ds(k * 16, 16)
                gidx_v[j, s] = lax.shift_right_logical(idx_ref[j, s], 7)
        copies = [pltpu.async_copy(
            v_hbm.at[gidx_v.at[j]], rows_v.at[pl.ds(j * _CHUNK, _CHUNK)], sem)
            for j in range(_NCH)]
        for c in copies:
            c.wait()

        def group(g, carry):
            vi = idx_ref[g >> 3, pl.ds((g & 7) * 16, 16)]
            lane = vi & 127
            val = plsc.load_gather(rows_v, [iota16 + g * 16, lane])
            s = pl.ds(g * 16, 16)
            if first:
                out_v[s] = val + bias
            else:
                out_v[s] = out_v[s] + val
            return carry

        lax.fori_loop(0, _BPW // 16, group, 0)

    one_table(vu_hbm, uidx_v, True)
    one_table(vi_hbm, iidx_v, False)

    pltpu.sync_copy(out_v, out_hbm.at[pl.ds(base, _BPW)])


@jax.jit
def _ncf(user_indices, item_indices, user_Tt, item_Tt, user_A, item_A,
         W_aff, params):
    grid = pl.cdiv(_N, _W)
    vu, vi = pl.pallas_call(
        _mv_body,
        grid=(grid,),
        in_specs=[
            pl.BlockSpec((_D, _W), lambda g: (0, g)),
            pl.BlockSpec((_D, _W), lambda g: (0, g)),
            pl.BlockSpec((_D, _LAT), lambda g: (0, 0)),
            pl.BlockSpec((_D, _LAT), lambda g: (0, 0)),
            pl.BlockSpec((2 * _LAT, 1), lambda g: (0, 0)),
        ],
        out_specs=[
            pl.BlockSpec((_W // 128, 128), lambda g: (g, 0)),
            pl.BlockSpec((_W // 128, 128), lambda g: (g, 0)),
        ],
        out_shape=[
            jax.ShapeDtypeStruct((_VROWS, 128), jnp.float32),
            jax.ShapeDtypeStruct((_VROWS, 128), jnp.float32),
        ],
        scratch_shapes=[pltpu.VMEM((2, _D), jnp.float32)],
    )(user_Tt, item_Tt, user_A, item_A, W_aff)

    run = pl.kernel(
        _gather_body,
        out_type=jax.ShapeDtypeStruct((_B,), jnp.float32),
        mesh=plsc.VectorSubcoreMesh(core_axis_name="c", subcore_axis_name="s"),
        compiler_params=pltpu.CompilerParams(needs_layout_passes=False,
                                             use_tc_tiling_on_sc=True),
        scratch_types=[
            pltpu.VMEM((_NCH, _CHUNK), jnp.int32),
            pltpu.VMEM((_NCH, _CHUNK), jnp.int32),
            pltpu.VMEM((_NCH, _CHUNK), jnp.int32),
            pltpu.VMEM((_BPW, 128), jnp.float32),
            pltpu.VMEM((264,), jnp.float32),
            pltpu.VMEM((_BPW,), jnp.float32),
            pltpu.SemaphoreType.DMA,
        ],
    )
    return run(user_indices, item_indices, vu, vi, params)


def kernel(user_indices, item_indices, user_T, item_T, user_A, item_A,
           W_aff, b_aff):
    params = jnp.concatenate([W_aff.reshape(-1), b_aff.reshape(-1),
                              jnp.zeros((7,), jnp.float32)])
    out = _ncf(user_indices.astype(jnp.int32), item_indices.astype(jnp.int32),
               user_T.T, item_T.T, user_A, item_A, W_aff, params)
    return out.reshape(_B, 1)


# speedup vs baseline: 6.3330x; 1.0097x over previous
import functools

import jax
import jax.numpy as jnp
from jax import lax
from jax.experimental import pallas as pl
from jax.experimental.pallas import tpu as pltpu
from jax.experimental.pallas import tpu_sc as plsc

_B = 16384
_N = 1000000
_D = 64
_LAT = 128
_NC = 2
_NS = 16
_NW = _NC * _NS
_BPW = _B // _NW
_CHUNK = 128
_NCH = _BPW // _CHUNK
_W = 32768
_VROWS = 8192

_HI = jax.lax.Precision.HIGHEST


def _mv_body(uT_ref, iT_ref, uA_ref, iA_ref, W_ref, vu_ref, vi_ref, w_s):
    @pl.when(pl.program_id(0) == 0)
    def _():
        w = W_ref[...]
        wu = jax.lax.dot_general(uA_ref[...], w[0:_LAT, :],
                                 (((1,), (0,)), ((), ())), precision=_HI)
        wi = jax.lax.dot_general(iA_ref[...], w[_LAT:2 * _LAT, :],
                                 (((1,), (0,)), ((), ())), precision=_HI)
        w_s[0:1, :] = wu.reshape(1, _D)
        w_s[1:2, :] = wi.reshape(1, _D)

    pu = jax.lax.dot_general(w_s[0:1, :], uT_ref[...],
                             (((1,), (0,)), ((), ())))
    pi = jax.lax.dot_general(w_s[1:2, :], iT_ref[...],
                             (((1,), (0,)), ((), ())))
    vu_ref[...] = pu.reshape(_W // 128, 128)
    vi_ref[...] = pi.reshape(_W // 128, 128)


def _gather_body(uidx_hbm, iidx_hbm, vu_hbm, vi_hbm, par_hbm, out_hbm,
                 uidx_v, iidx_v, gidx_v, rows_v, par_v, out_v, sem):
    wid = lax.axis_index("s") * _NC + lax.axis_index("c")
    base = wid * _BPW

    for j in range(_NCH):
        pltpu.sync_copy(uidx_hbm.at[pl.ds(base + j * _CHUNK, _CHUNK)],
                        uidx_v.at[j])
        pltpu.sync_copy(iidx_hbm.at[pl.ds(base + j * _CHUNK, _CHUNK)],
                        iidx_v.at[j])
    pltpu.sync_copy(par_hbm, par_v)

    bias = plsc.load_gather(par_v, [jnp.full((16,), 2 * _LAT, jnp.int32)])
    iota16 = lax.iota(jnp.int32, 16)

    def one_table(v_hbm, idx_ref, first):
        for j in range(_NCH):
            for k in range(_CHUNK // 16):
                s = pl.ds(k * 16, 16)
                gidx_v[j, s] = lax.shift_right_logical(idx_ref[j, s], 7)
        copies = [pltpu.async_copy(
            v_hbm.at[gidx_v.at[j]], rows_v.at[pl.ds(j * _CHUNK, _CHUNK)], sem)
            for j in range(_NCH)]
        for c in copies:
            c.wait()

        def group(g, carry):
            vi = idx_ref[g >> 3, pl.ds((g & 7) * 16, 16)]
            lane = vi & 127
            val = plsc.load_gather(rows_v, [iota16 + g * 16, lane])
            s = pl.ds(g * 16, 16)
            if first:
                out_v[s] = val + bias
            else:
                out_v[s] = out_v[s] + val
            return carry

        lax.fori_loop(0, _BPW // 16, group, 0)

    one_table(vu_hbm, uidx_v, True)
    one_table(vi_hbm, iidx_v, False)

    pltpu.sync_copy(out_v, out_hbm.at[pl.ds(base, _BPW)])


@jax.jit
def _ncf(user_indices, item_indices, user_Tt, item_Tt, user_A, item_A,
         W_aff, params):
    grid = pl.cdiv(_N, _W)
    vu, vi = pl.pallas_call(
        _mv_body,
        grid=(grid,),
        in_specs=[
            pl.BlockSpec((_D, _W), lambda g: (0, g)),
            pl.BlockSpec((_D, _W), lambda g: (0, g)),
            pl.BlockSpec((_D, _LAT), lambda g: (0, 0)),
            pl.BlockSpec((_D, _LAT), lambda g: (0, 0)),
            pl.BlockSpec((2 * _LAT, 1), lambda g: (0, 0)),
        ],
        out_specs=[
            pl.BlockSpec((_W // 128, 128), lambda g: (g, 0)),
            pl.BlockSpec((_W // 128, 128), lambda g: (g, 0)),
        ],
        out_shape=[
            jax.ShapeDtypeStruct((_VROWS, 128), jnp.float32),
            jax.ShapeDtypeStruct((_VROWS, 128), jnp.float32),
        ],
        scratch_shapes=[pltpu.VMEM((2, _D), jnp.float32)],
    )(user_Tt, item_Tt, user_A, item_A, W_aff)

    run = pl.kernel(
        _gather_body,
        out_type=jax.ShapeDtypeStruct((_B,), jnp.float32),
        mesh=plsc.VectorSubcoreMesh(core_axis_name="c", subcore_axis_name="s"),
        compiler_params=pltpu.CompilerParams(needs_layout_passes=False,
                                             use_tc_tiling_on_sc=True),
        scratch_types=[
            pltpu.VMEM((_NCH, _CHUNK), jnp.int32),
            pltpu.VMEM((_NCH, _CHUNK), jnp.int32),
            pltpu.VMEM((_NCH, _CHUNK), jnp.int32),
            pltpu.VMEM((_BPW, 128), jnp.float32),
            pltpu.VMEM((264,), jnp.float32),
            pltpu.VMEM((_BPW,), jnp.float32),
            pltpu.SemaphoreType.DMA,
        ],
    )
    return run(user_indices, item_indices, vu, vi, params)


def kernel(user_indices, item_indices, user_T, item_T, user_A, item_A,
           W_aff, b_aff):
    params = jnp.concatenate([W_aff.reshape(-1), b_aff.reshape(-1),
                              jnp.zeros((7,), jnp.float32)])
    out = _ncf(user_indices.astype(jnp.int32), item_indices.astype(jnp.int32),
               user_T.T, item_T.T, user_A, item_A, W_aff, params)
    return out.reshape(_B, 1)


# speedup vs baseline: 6.3351x; 1.0003x over previous
import functools

import jax
import jax.numpy as jnp
from jax import lax
from jax.experimental import pallas as pl
from jax.experimental.pallas import tpu as pltpu
from jax.experimental.pallas import tpu_sc as plsc

_B = 16384
_N = 1000000
_D = 64
_LAT = 128
_NC = 2
_NS = 16
_NW = _NC * _NS
_BPW = _B // _NW
_CHUNK = 128
_NCH = _BPW // _CHUNK
_W = 32768
_VROWS = 8192

_HI = jax.lax.Precision.HIGHEST


def _mv_body(uT_ref, iT_ref, uA_ref, iA_ref, W_ref, vu_ref, vi_ref, w_s):
    @pl.when(pl.program_id(0) == 0)
    def _():
        w = W_ref[...]
        wu = jax.lax.dot_general(uA_ref[...], w[0:_LAT, :],
                                 (((1,), (0,)), ((), ())), precision=_HI)
        wi = jax.lax.dot_general(iA_ref[...], w[_LAT:2 * _LAT, :],
                                 (((1,), (0,)), ((), ())), precision=_HI)
        w_s[0:1, :] = wu.reshape(1, _D)
        w_s[1:2, :] = wi.reshape(1, _D)

    pu = jax.lax.dot_general(w_s[0:1, :], uT_ref[...],
                             (((1,), (0,)), ((), ())))
    pi = jax.lax.dot_general(w_s[1:2, :], iT_ref[...],
                             (((1,), (0,)), ((), ())))
    vu_ref[...] = pu.reshape(_W // 128, 128)
    vi_ref[...] = pi.reshape(_W // 128, 128)


def _gather_body(uidx_hbm, iidx_hbm, vu_hbm, vi_hbm, par_hbm, out_hbm,
                 uidx_v, iidx_v, gu_v, gi_v, rowsA, rowsB, par_v, out_v,
                 semA, semB, semS):
    wid = lax.axis_index("s") * _NC + lax.axis_index("c")
    base = wid * _BPW

    stage = [pltpu.async_copy(
        uidx_hbm.at[pl.ds(base + j * _CHUNK, _CHUNK)], uidx_v.at[j], semS)
        for j in range(_NCH)]
    stage += [pltpu.async_copy(
        iidx_hbm.at[pl.ds(base + j * _CHUNK, _CHUNK)], iidx_v.at[j], semS)
        for j in range(_NCH)]
    stage.append(pltpu.async_copy(par_hbm, par_v, semS))
    for c in stage:
        c.wait()

    for j in range(_NCH):
        for k in range(_CHUNK // 16):
            s = pl.ds(k * 16, 16)
            gu_v[j, s] = lax.shift_right_logical(uidx_v[j, s], 7)
            gi_v[j, s] = lax.shift_right_logical(iidx_v[j, s], 7)

    half = [pl.ds(0, _CHUNK), pl.ds(_CHUNK, _CHUNK)]

    def fire(v_hbm, g_ref, h, buf, sem):
        return [pltpu.async_copy(
            v_hbm.at[g_ref.at[2 * h + j]], buf.at[half[j]], sem)
            for j in range(2)]

    bias = plsc.load_gather(par_v, [jnp.full((16,), 2 * _LAT, jnp.int32)])
    iota16 = lax.iota(jnp.int32, 16)

    def extract(idx_ref, h, buf, first):
        def group(g, carry):
            vi = idx_ref[(h * 16 + g) >> 3, pl.ds(((h * 16 + g) & 7) * 16, 16)]
            lane = vi & 127
            val = plsc.load_gather(buf, [iota16 + g * 16, lane])
            s = pl.ds(h * _BPW // 2 + g * 16, 16)
            if first:
                out_v[s] = val + bias
            else:
                out_v[s] = out_v[s] + val
            return carry
        lax.fori_loop(0, _BPW // 32, group, 0)

    cA = fire(vu_hbm, gu_v, 0, rowsA, semA)
    cB = fire(vu_hbm, gu_v, 1, rowsB, semB)
    for c in cA:
        c.wait()
    extract(uidx_v, 0, rowsA, True)
    cA = fire(vi_hbm, gi_v, 0, rowsA, semA)
    for c in cB:
        c.wait()
    extract(uidx_v, 1, rowsB, True)
    cB = fire(vi_hbm, gi_v, 1, rowsB, semB)
    for c in cA:
        c.wait()
    extract(iidx_v, 0, rowsA, False)
    for c in cB:
        c.wait()
    extract(iidx_v, 1, rowsB, False)

    pltpu.sync_copy(out_v, out_hbm.at[pl.ds(base, _BPW)])


@jax.jit
def _ncf(user_indices, item_indices, user_Tt, item_Tt, user_A, item_A,
         W_aff, params):
    grid = pl.cdiv(_N, _W)
    vu, vi = pl.pallas_call(
        _mv_body,
        grid=(grid,),
        in_specs=[
            pl.BlockSpec((_D, _W), lambda g: (0, g)),
            pl.BlockSpec((_D, _W), lambda g: (0, g)),
            pl.BlockSpec((_D, _LAT), lambda g: (0, 0)),
            pl.BlockSpec((_D, _LAT), lambda g: (0, 0)),
            pl.BlockSpec((2 * _LAT, 1), lambda g: (0, 0)),
        ],
        out_specs=[
            pl.BlockSpec((_W // 128, 128), lambda g: (g, 0)),
            pl.BlockSpec((_W // 128, 128), lambda g: (g, 0)),
        ],
        out_shape=[
            jax.ShapeDtypeStruct((_VROWS, 128), jnp.float32),
            jax.ShapeDtypeStruct((_VROWS, 128), jnp.float32),
        ],
        scratch_shapes=[pltpu.VMEM((2, _D), jnp.float32)],
    )(user_Tt, item_Tt, user_A, item_A, W_aff)

    run = pl.kernel(
        _gather_body,
        out_type=jax.ShapeDtypeStruct((_B,), jnp.float32),
        mesh=plsc.VectorSubcoreMesh(core_axis_name="c", subcore_axis_name="s"),
        compiler_params=pltpu.CompilerParams(needs_layout_passes=False,
                                             use_tc_tiling_on_sc=True),
        scratch_types=[
            pltpu.VMEM((_NCH, _CHUNK), jnp.int32),
            pltpu.VMEM((_NCH, _CHUNK), jnp.int32),
            pltpu.VMEM((_NCH, _CHUNK), jnp.int32),
            pltpu.VMEM((_NCH, _CHUNK), jnp.int32),
            pltpu.VMEM((_BPW // 2, 128), jnp.float32),
            pltpu.VMEM((_BPW // 2, 128), jnp.float32),
            pltpu.VMEM((264,), jnp.float32),
            pltpu.VMEM((_BPW,), jnp.float32),
            pltpu.SemaphoreType.DMA,
            pltpu.SemaphoreType.DMA,
            pltpu.SemaphoreType.DMA,
        ],
    )
    return run(user_indices, item_indices, vu, vi, params)


def kernel(user_indices, item_indices, user_T, item_T, user_A, item_A,
           W_aff, b_aff):
    params = jnp.concatenate([W_aff.reshape(-1), b_aff.reshape(-1),
                              jnp.zeros((7,), jnp.float32)])
    out = _ncf(user_indices.astype(jnp.int32), item_indices.astype(jnp.int32),
               user_T.T, item_T.T, user_A, item_A, W_aff, params)
    return out.reshape(_B, 1)
